# baseline (speedup 1.0000x reference)
import functools

import jax
import jax.numpy as jnp
from jax import lax
from jax.experimental import pallas as pl
from jax.experimental.pallas import tpu as pltpu
from jax.experimental.pallas import tpu_sc as plsc

VOCAB = 1000000
D = 64
B = 1024
S = 200
F = 4

NC = 2
NS = 16
NW = NC * NS

T = B * S
T_W = T // NW
CHUNK = 128
NCH = T_W // CHUNK


def _gather_body(i0, i1, i2, i3, table_hbm, out_hbm,
                 v0, v1, v2, v3, rows0, rows1, gsem, osem):
    wid = lax.axis_index("s") * NC + lax.axis_index("c")
    base = wid * T_W
    idx_v = (v0, v1, v2, v3)
    for f, ih in enumerate((i0, i1, i2, i3)):
        pltpu.sync_copy(ih.at[wid], idx_v[f])
    bufs = (rows0, rows1)

    def fill(jb, buf):
        copies = [
            pltpu.async_copy(
                table_hbm.at[idx_v[f].at[jb]],
                buf.at[f],
                gsem,
            )
            for f in range(F)
        ]
        for cpy in copies:
            cpy.wait()
        for f in range(F):
            pltpu.async_copy(
                buf.at[f],
                out_hbm.at[pl.ds(base + jb * CHUNK, CHUNK), f],
                osem,
            )

    def wait_wb(buf):
        for f in range(F):
            pltpu.make_async_copy(
                buf.at[f], out_hbm.at[pl.ds(base, CHUNK), f], osem
            ).wait()

    for p in range(2):
        fill(p, bufs[p])

    def blk2(j2, _):
        for p in range(2):
            wait_wb(bufs[p])
            fill(j2 * 2 + p, bufs[p])
        return 0

    lax.fori_loop(1, NCH // 2, blk2, 0)

    for p in range(2):
        wait_wb(bufs[p])


_gather = functools.partial(
    pl.kernel,
    mesh=plsc.VectorSubcoreMesh(core_axis_name="c", subcore_axis_name="s"),
    out_type=jax.ShapeDtypeStruct((T, F, D), jnp.float32),
    compiler_params=pltpu.CompilerParams(use_tc_tiling_on_sc=False),
    scratch_types=[
        pltpu.VMEM((NCH, CHUNK), jnp.int32),
        pltpu.VMEM((NCH, CHUNK), jnp.int32),
        pltpu.VMEM((NCH, CHUNK), jnp.int32),
        pltpu.VMEM((NCH, CHUNK), jnp.int32),
        pltpu.VMEM((F, CHUNK, D), jnp.float32),
        pltpu.VMEM((F, CHUNK, D), jnp.float32),
        pltpu.SemaphoreType.DMA,
        pltpu.SemaphoreType.DMA,
    ],
)(_gather_body)


def kernel(table, idx_f0, idx_f1, idx_f2, idx_f3, seq_lens):
    del seq_lens
    shaped = [
        a.reshape(NW, NCH, CHUNK) for a in (idx_f0, idx_f1, idx_f2, idx_f3)
    ]
    out = _gather(*shaped, table)
    return out.reshape(B, S, F * D)


# speedup vs baseline: 1.0589x; 1.0589x over previous
import functools

import jax
import jax.numpy as jnp
from jax import lax
from jax.experimental import pallas as pl
from jax.experimental.pallas import tpu as pltpu
from jax.experimental.pallas import tpu_sc as plsc

VOCAB = 1000000
D = 64
B = 1024
S = 200
F = 4

NC = 2
NS = 16
NW = NC * NS

N = B * S * F
N_W = N // NW
CHUNK = 128
NCH = N_W // CHUNK
NBUF = 4
BLKR = NBUF * CHUNK
NBLK = NCH // NBUF


def _gather_body(idx_hbm, table_hbm, out_hbm, idx_v, rows0, rows1, gsem, osem):
    wid = lax.axis_index("s") * NC + lax.axis_index("c")
    base = wid * N_W
    table2 = table_hbm
    pltpu.sync_copy(idx_hbm.at[wid], idx_v)
    bufs = (rows0, rows1)

    def fill(jb, buf):
        copies = [
            pltpu.async_copy(
                table2.at[idx_v.at[jb * NBUF + b]],
                buf.at[pl.ds(b * CHUNK, CHUNK)],
                gsem,
            )
            for b in range(NBUF)
        ]
        for cpy in copies:
            cpy.wait()
        pltpu.async_copy(buf, out_hbm.at[pl.ds(base + jb * BLKR, BLKR)], osem)

    for p in range(2):
        fill(p, bufs[p])

    def blk2(j2, _):
        for p in range(2):
            pltpu.make_async_copy(
                bufs[p], out_hbm.at[pl.ds(base, BLKR)], osem
            ).wait()
            fill(j2 * 2 + p, bufs[p])
        return 0

    lax.fori_loop(1, NBLK // 2, blk2, 0)

    for p in range(2):
        pltpu.make_async_copy(
            bufs[p], out_hbm.at[pl.ds(base, BLKR)], osem
        ).wait()


_gather = functools.partial(
    pl.kernel,
    mesh=plsc.VectorSubcoreMesh(core_axis_name="c", subcore_axis_name="s"),
    out_type=jax.ShapeDtypeStruct((N, D), jnp.float32),
    compiler_params=pltpu.CompilerParams(use_tc_tiling_on_sc=False),
    scratch_types=[
        pltpu.VMEM((NCH, CHUNK), jnp.int32),
        pltpu.VMEM((BLKR, D), jnp.float32),
        pltpu.VMEM((BLKR, D), jnp.float32),
        pltpu.SemaphoreType.DMA,
        pltpu.SemaphoreType.DMA,
    ],
)(_gather_body)


def kernel(table, idx_f0, idx_f1, idx_f2, idx_f3, seq_lens):
    del seq_lens
    idx = jnp.stack(
        [idx_f0, idx_f1, idx_f2, idx_f3], axis=-1
    )
    idx = idx.reshape(B, S // 8, 8, 2, 2)
    idx = idx.transpose(0, 1, 3, 2, 4)
    idx = idx.reshape(NW, NCH, CHUNK)
    out = _gather(idx, table)
    out = out.reshape(B, S // 8, 2, 8, 2 * D)
    out = out.transpose(0, 1, 3, 2, 4)
    return out.reshape(B, S, F * D)
